# Initial kernel scaffold; baseline (speedup 1.0000x reference)
import jax
import jax.numpy as jnp
from jax.experimental import pallas as pl


def kernel(x_real, x_imag, W, b):
    raise NotImplementedError("write your pallas kernel here")


# speedup vs baseline: 1.4670x; 1.4670x over previous
import functools

import jax
import jax.numpy as jnp
from jax.experimental import pallas as pl
from jax.experimental.pallas import tpu as pltpu

B, S, D, E, TOPK = 4, 8192, 768, 64, 2
BS = B * S
BM = 512


def _gating_kernel(xr_ref, xi_ref, wa_ref, wp_ref, b_ref, probs_ref, idx_ref):
    xr = xr_ref[...]
    xi = xi_ref[...]
    amp = jnp.sqrt(xr * xr + xi * xi)
    phase = jnp.arctan2(xi, xr)
    scores = (
        jnp.dot(amp, wa_ref[...], preferred_element_type=jnp.float32)
        + jnp.dot(phase, wp_ref[...], preferred_element_type=jnp.float32)
        + b_ref[...]
    )

    lane = jax.lax.broadcasted_iota(jnp.int32, scores.shape, 1)
    m1 = jnp.max(scores, axis=-1, keepdims=True)
    i1 = jnp.min(jnp.where(scores == m1, lane, E), axis=-1, keepdims=True)
    masked = jnp.where(lane == i1, -jnp.inf, scores)
    m2 = jnp.max(masked, axis=-1, keepdims=True)
    i2 = jnp.min(jnp.where(masked == m2, lane, E), axis=-1, keepdims=True)

    e = jnp.exp(m2 - m1)
    p0 = 1.0 / (1.0 + e)
    probs_ref[:, 0:1] = p0
    probs_ref[:, 1:2] = 1.0 - p0
    idx_ref[:, 0:1] = i1
    idx_ref[:, 1:2] = i2


@jax.jit
def kernel(x_real, x_imag, W, b):
    xr = x_real.reshape(BS, D)
    xi = x_imag.reshape(BS, D)
    wa = W[:D]
    wp = W[D:]
    b2 = b.reshape(1, E)

    grid = (BS // BM,)
    probs, idx = pl.pallas_call(
        _gating_kernel,
        grid=grid,
        in_specs=[
            pl.BlockSpec((BM, D), lambda i: (i, 0)),
            pl.BlockSpec((BM, D), lambda i: (i, 0)),
            pl.BlockSpec((D, E), lambda i: (0, 0)),
            pl.BlockSpec((D, E), lambda i: (0, 0)),
            pl.BlockSpec((1, E), lambda i: (0, 0)),
        ],
        out_specs=[
            pl.BlockSpec((BM, TOPK), lambda i: (i, 0)),
            pl.BlockSpec((BM, TOPK), lambda i: (i, 0)),
        ],
        out_shape=[
            jax.ShapeDtypeStruct((BS, TOPK), jnp.float32),
            jax.ShapeDtypeStruct((BS, TOPK), jnp.int32),
        ],
        compiler_params=pltpu.CompilerParams(
            dimension_semantics=("arbitrary",),
        ),
    )(xr, xi, wa, wp, b2)

    return probs.reshape(B, S, TOPK), idx.reshape(B, S, TOPK)


# speedup vs baseline: 1.8583x; 1.2667x over previous
import functools

import jax
import jax.numpy as jnp
from jax.experimental import pallas as pl
from jax.experimental.pallas import tpu as pltpu

B, S, D, E, TOPK = 4, 8192, 768, 64, 2
BS = B * S
BM = 512


_C0 = 0.999995508
_C1 = -0.33298865
_C2 = 0.195589143
_C3 = -0.121109628
_C4 = 0.0573306763
_C5 = -0.0134222103
_HALF_PI = 1.5707963267948966
_PI = 3.141592653589793


def _fast_atan2(y, x):
    ax = jnp.abs(x)
    ay = jnp.abs(y)
    mx = jnp.maximum(ax, ay)
    mn = jnp.minimum(ax, ay)
    t = mn / jnp.maximum(mx, 1e-35)
    z = t * t
    p = ((((_C5 * z + _C4) * z + _C3) * z + _C2) * z + _C1) * z + _C0
    r = t * p
    r = jnp.where(ay > ax, _HALF_PI - r, r)
    r = jnp.where(x < 0, _PI - r, r)
    return jnp.where(y < 0, -r, r)


def _gating_kernel(xr_ref, xi_ref, wa_ref, wp_ref, b_ref, probs_ref, idx_ref):
    xr = xr_ref[...]
    xi = xi_ref[...]
    amp = jnp.sqrt(xr * xr + xi * xi)
    phase = _fast_atan2(xi, xr)
    scores = (
        jnp.dot(amp, wa_ref[...], preferred_element_type=jnp.float32)
        + jnp.dot(phase, wp_ref[...], preferred_element_type=jnp.float32)
        + b_ref[...]
    )

    lane = jax.lax.broadcasted_iota(jnp.int32, scores.shape, 1)
    m1 = jnp.max(scores, axis=-1, keepdims=True)
    i1 = jnp.min(jnp.where(scores == m1, lane, E), axis=-1, keepdims=True)
    masked = jnp.where(lane == i1, -jnp.inf, scores)
    m2 = jnp.max(masked, axis=-1, keepdims=True)
    i2 = jnp.min(jnp.where(masked == m2, lane, E), axis=-1, keepdims=True)

    e = jnp.exp(m2 - m1)
    p0 = 1.0 / (1.0 + e)
    probs_ref[:, 0:1] = p0
    probs_ref[:, 1:2] = 1.0 - p0
    idx_ref[:, 0:1] = i1
    idx_ref[:, 1:2] = i2


@jax.jit
def kernel(x_real, x_imag, W, b):
    xr = x_real.reshape(BS, D)
    xi = x_imag.reshape(BS, D)
    wa = W[:D]
    wp = W[D:]
    b2 = b.reshape(1, E)

    grid = (BS // BM,)
    probs, idx = pl.pallas_call(
        _gating_kernel,
        grid=grid,
        in_specs=[
            pl.BlockSpec((BM, D), lambda i: (i, 0)),
            pl.BlockSpec((BM, D), lambda i: (i, 0)),
            pl.BlockSpec((D, E), lambda i: (0, 0)),
            pl.BlockSpec((D, E), lambda i: (0, 0)),
            pl.BlockSpec((1, E), lambda i: (0, 0)),
        ],
        out_specs=[
            pl.BlockSpec((BM, TOPK), lambda i: (i, 0)),
            pl.BlockSpec((BM, TOPK), lambda i: (i, 0)),
        ],
        out_shape=[
            jax.ShapeDtypeStruct((BS, TOPK), jnp.float32),
            jax.ShapeDtypeStruct((BS, TOPK), jnp.int32),
        ],
        compiler_params=pltpu.CompilerParams(
            dimension_semantics=("arbitrary",),
        ),
    )(xr, xi, wa, wp, b2)

    return probs.reshape(B, S, TOPK), idx.reshape(B, S, TOPK)
